# baseline (device time: 31762 ns/iter reference)
import jax
import jax.numpy as jnp
from jax import lax
from jax.experimental import pallas as pl
from jax.experimental.pallas import tpu as pltpu

N_DEV = 8
E_PER = 2
R_PAD = 8
NEG = -1e30


def kernel(x, router, W1, W2):
    t_per, d = x.shape
    T = N_DEV * t_per
    f = W1.shape[2]

    rT = jnp.zeros((R_PAD, d), x.dtype).at[:E_PER, :].set(router.T)

    def body(x_ref, rT_ref, w1_ref, w2_ref, out_ref,
             xg_ref, rg_ref, acc_ref, rs_ref,
             x_send, x_recv, r_send, r_recv, rs_send, rs_recv):
        my = lax.axis_index("i")

        barrier = pltpu.get_barrier_semaphore()
        for o in range(1, N_DEV):
            pl.semaphore_signal(
                barrier, inc=1,
                device_id=(lax.rem(my + o, N_DEV),),
                device_id_type=pl.DeviceIdType.MESH,
            )
        pl.semaphore_wait(barrier, N_DEV - 1)

        xg_ref[pl.ds(my * t_per, t_per), :] = x_ref[...]
        rg_ref[pl.ds(my * R_PAD, R_PAD), :] = rT_ref[...]

        x_sends, r_sends = [], []
        for o in range(1, N_DEV):
            dst = lax.rem(my + o, N_DEV)
            rdma = pltpu.make_async_remote_copy(
                src_ref=x_ref,
                dst_ref=xg_ref.at[pl.ds(my * t_per, t_per), :],
                send_sem=x_send.at[dst],
                recv_sem=x_recv.at[my],
                device_id=(dst,),
                device_id_type=pl.DeviceIdType.MESH,
            )
            rdma.start()
            x_sends.append(rdma)
            rdma = pltpu.make_async_remote_copy(
                src_ref=rT_ref,
                dst_ref=rg_ref.at[pl.ds(my * R_PAD, R_PAD), :],
                send_sem=r_send.at[dst],
                recv_sem=r_recv.at[my],
                device_id=(dst,),
                device_id_type=pl.DeviceIdType.MESH,
            )
            rdma.start()
            r_sends.append(rdma)

        for o in range(1, N_DEV):
            src = lax.rem(my + o, N_DEV)
            pltpu.make_async_remote_copy(
                src_ref=x_ref,
                dst_ref=xg_ref.at[pl.ds(src * t_per, t_per), :],
                send_sem=x_send.at[src],
                recv_sem=x_recv.at[src],
                device_id=(src,),
                device_id_type=pl.DeviceIdType.MESH,
            ).wait_recv()
            pltpu.make_async_remote_copy(
                src_ref=rT_ref,
                dst_ref=rg_ref.at[pl.ds(src * R_PAD, R_PAD), :],
                send_sem=r_send.at[src],
                recv_sem=r_recv.at[src],
                device_id=(src,),
                device_id_type=pl.DeviceIdType.MESH,
            ).wait_recv()

        X = xg_ref[...]
        RT = rg_ref[...]
        gp = lax.dot_general(
            X, RT, (((1,), (1,)), ((), ())),
            preferred_element_type=jnp.float32,
        )
        col = lax.broadcasted_iota(jnp.int32, (T, N_DEV * R_PAD), 1)
        g = jnp.where(lax.rem(col, R_PAD) < E_PER, gp, NEG)
        m1 = jnp.max(g, axis=1, keepdims=True)
        a1 = jnp.min(jnp.where(g == m1, col, N_DEV * R_PAD), 1, keepdims=True)
        g2 = jnp.where(col == a1, NEG, g)
        m2 = jnp.max(g2, axis=1, keepdims=True)
        a2 = jnp.min(jnp.where(g2 == m2, col, N_DEV * R_PAD), 1, keepdims=True)
        w_top = 1.0 / (1.0 + jnp.exp(m2 - m1))
        w_sec = 1.0 - w_top

        acc = jnp.zeros((T, d), jnp.float32)
        for e in range(E_PER):
            pid = my * R_PAD + e
            wt = (jnp.where(a1 == pid, w_top, 0.0)
                  + jnp.where(a2 == pid, w_sec, 0.0))
            h = jnp.maximum(
                jnp.dot(X, w1_ref[e], preferred_element_type=jnp.float32),
                0.0,
            )
            y = jnp.dot(h, w2_ref[e], preferred_element_type=jnp.float32)
            acc = acc + y * wt
        acc_ref[...] = acc

        for rdma in x_sends + r_sends:
            rdma.wait_send()

        rs_sends = []
        for o in range(1, N_DEV):
            dst = lax.rem(my + o, N_DEV)
            rdma = pltpu.make_async_remote_copy(
                src_ref=acc_ref.at[pl.ds(dst * t_per, t_per), :],
                dst_ref=rs_ref.at[pl.ds(my * t_per, t_per), :],
                send_sem=rs_send.at[dst],
                recv_sem=rs_recv.at[my],
                device_id=(dst,),
                device_id_type=pl.DeviceIdType.MESH,
            )
            rdma.start()
            rs_sends.append(rdma)

        total = acc_ref[pl.ds(my * t_per, t_per), :]
        for o in range(1, N_DEV):
            src = lax.rem(my + o, N_DEV)
            pltpu.make_async_remote_copy(
                src_ref=acc_ref.at[pl.ds(src * t_per, t_per), :],
                dst_ref=rs_ref.at[pl.ds(src * t_per, t_per), :],
                send_sem=rs_send.at[src],
                recv_sem=rs_recv.at[src],
                device_id=(src,),
                device_id_type=pl.DeviceIdType.MESH,
            ).wait_recv()
            total = total + rs_ref[pl.ds(src * t_per, t_per), :]
        out_ref[...] = total

        for rdma in rs_sends:
            rdma.wait_send()

    return pl.pallas_call(
        body,
        out_shape=jax.ShapeDtypeStruct((t_per, d), jnp.float32),
        in_specs=[pl.BlockSpec(memory_space=pltpu.VMEM)] * 4,
        out_specs=pl.BlockSpec(memory_space=pltpu.VMEM),
        scratch_shapes=[
            pltpu.VMEM((T, d), jnp.float32),
            pltpu.VMEM((N_DEV * R_PAD, d), jnp.float32),
            pltpu.VMEM((T, d), jnp.float32),
            pltpu.VMEM((T, d), jnp.float32),
            pltpu.SemaphoreType.DMA((N_DEV,)),
            pltpu.SemaphoreType.DMA((N_DEV,)),
            pltpu.SemaphoreType.DMA((N_DEV,)),
            pltpu.SemaphoreType.DMA((N_DEV,)),
            pltpu.SemaphoreType.DMA((N_DEV,)),
            pltpu.SemaphoreType.DMA((N_DEV,)),
        ],
        compiler_params=pltpu.CompilerParams(collective_id=0),
    )(x, rT, W1, W2)


# device time: 28764 ns/iter; 1.1042x vs baseline; 1.1042x over previous
import jax
import jax.numpy as jnp
from jax import lax
from jax.experimental import pallas as pl
from jax.experimental.pallas import tpu as pltpu

N_DEV = 8
E_PER = 2
R_PAD = 8
NEG = -1e30


def kernel(x, router, W1, W2):
    t_per, d = x.shape
    T = N_DEV * t_per

    rT = jnp.zeros((R_PAD, d), x.dtype).at[:E_PER, :].set(router.T)

    def body(x_ref, rT_ref, w1_ref, w2_ref, out_ref,
             xg_ref, rg_ref, acc_ref, rs_ref,
             x_send, x_recv, r_send, r_recv, rs_send, rs_recv):
        my = lax.axis_index("i")

        barrier = pltpu.get_barrier_semaphore()
        for o in range(1, N_DEV):
            pl.semaphore_signal(
                barrier, inc=1,
                device_id=(lax.rem(my + o, N_DEV),),
                device_id_type=pl.DeviceIdType.MESH,
            )
        pl.semaphore_wait(barrier, N_DEV - 1)

        r_sends, x_sends = [], []
        for o in range(1, N_DEV):
            dst = lax.rem(my + o, N_DEV)
            rdma = pltpu.make_async_remote_copy(
                src_ref=rT_ref,
                dst_ref=rg_ref.at[pl.ds(my * R_PAD, R_PAD), :],
                send_sem=r_send.at[dst],
                recv_sem=r_recv.at[my],
                device_id=(dst,),
                device_id_type=pl.DeviceIdType.MESH,
            )
            rdma.start()
            r_sends.append(rdma)
        for o in range(1, N_DEV):
            dst = lax.rem(my + o, N_DEV)
            rdma = pltpu.make_async_remote_copy(
                src_ref=x_ref,
                dst_ref=xg_ref.at[pl.ds(my * t_per, t_per), :],
                send_sem=x_send.at[dst],
                recv_sem=x_recv.at[my],
                device_id=(dst,),
                device_id_type=pl.DeviceIdType.MESH,
            )
            rdma.start()
            x_sends.append(rdma)

        rg_ref[pl.ds(my * R_PAD, R_PAD), :] = rT_ref[...]
        for o in range(1, N_DEV):
            src = lax.rem(my - o + N_DEV, N_DEV)
            pltpu.make_async_remote_copy(
                src_ref=rT_ref,
                dst_ref=rg_ref.at[pl.ds(src * R_PAD, R_PAD), :],
                send_sem=r_send.at[src],
                recv_sem=r_recv.at[src],
                device_id=(src,),
                device_id_type=pl.DeviceIdType.MESH,
            ).wait_recv()
        RT = rg_ref[...]

        col = lax.broadcasted_iota(jnp.int32, (t_per, N_DEV * R_PAD), 1)

        def chunk_out(Xc):
            gp = lax.dot_general(
                Xc, RT, (((1,), (1,)), ((), ())),
                preferred_element_type=jnp.float32,
            )
            g = jnp.where(lax.rem(col, R_PAD) < E_PER, gp, NEG)
            m1 = jnp.max(g, axis=1, keepdims=True)
            a1 = jnp.min(jnp.where(g == m1, col, N_DEV * R_PAD), 1, keepdims=True)
            g2 = jnp.where(col == a1, NEG, g)
            m2 = jnp.max(g2, axis=1, keepdims=True)
            a2 = jnp.min(jnp.where(g2 == m2, col, N_DEV * R_PAD), 1, keepdims=True)
            w_top = 1.0 / (1.0 + jnp.exp(m2 - m1))
            w_sec = 1.0 - w_top
            y = jnp.zeros((t_per, d), jnp.float32)
            for e in range(E_PER):
                pid = my * R_PAD + e
                wt = (jnp.where(a1 == pid, w_top, 0.0)
                      + jnp.where(a2 == pid, w_sec, 0.0))
                h = jnp.maximum(
                    jnp.dot(Xc, w1_ref[e], preferred_element_type=jnp.float32),
                    0.0,
                )
                y = y + jnp.dot(h, w2_ref[e],
                                preferred_element_type=jnp.float32) * wt
            return y

        total = chunk_out(x_ref[...])

        rs_sends = []
        for o in range(1, N_DEV):
            src = lax.rem(my - o + N_DEV, N_DEV)
            pltpu.make_async_remote_copy(
                src_ref=x_ref,
                dst_ref=xg_ref.at[pl.ds(src * t_per, t_per), :],
                send_sem=x_send.at[src],
                recv_sem=x_recv.at[src],
                device_id=(src,),
                device_id_type=pl.DeviceIdType.MESH,
            ).wait_recv()
            yc = chunk_out(xg_ref[pl.ds(src * t_per, t_per), :])
            acc_ref[pl.ds(src * t_per, t_per), :] = yc
            rdma = pltpu.make_async_remote_copy(
                src_ref=acc_ref.at[pl.ds(src * t_per, t_per), :],
                dst_ref=rs_ref.at[pl.ds(my * t_per, t_per), :],
                send_sem=rs_send.at[src],
                recv_sem=rs_recv.at[my],
                device_id=(src,),
                device_id_type=pl.DeviceIdType.MESH,
            )
            rdma.start()
            rs_sends.append(rdma)

        for o in range(1, N_DEV):
            src = lax.rem(my + o, N_DEV)
            pltpu.make_async_remote_copy(
                src_ref=x_ref,
                dst_ref=rs_ref.at[pl.ds(src * t_per, t_per), :],
                send_sem=rs_send.at[src],
                recv_sem=rs_recv.at[src],
                device_id=(src,),
                device_id_type=pl.DeviceIdType.MESH,
            ).wait_recv()
            total = total + rs_ref[pl.ds(src * t_per, t_per), :]
        out_ref[...] = total

        for rdma in r_sends + x_sends + rs_sends:
            rdma.wait_send()

    return pl.pallas_call(
        body,
        out_shape=jax.ShapeDtypeStruct((t_per, d), jnp.float32),
        in_specs=[pl.BlockSpec(memory_space=pltpu.VMEM)] * 4,
        out_specs=pl.BlockSpec(memory_space=pltpu.VMEM),
        scratch_shapes=[
            pltpu.VMEM((T, d), jnp.float32),
            pltpu.VMEM((N_DEV * R_PAD, d), jnp.float32),
            pltpu.VMEM((T, d), jnp.float32),
            pltpu.VMEM((T, d), jnp.float32),
            pltpu.SemaphoreType.DMA((N_DEV,)),
            pltpu.SemaphoreType.DMA((N_DEV,)),
            pltpu.SemaphoreType.DMA((N_DEV,)),
            pltpu.SemaphoreType.DMA((N_DEV,)),
            pltpu.SemaphoreType.DMA((N_DEV,)),
            pltpu.SemaphoreType.DMA((N_DEV,)),
        ],
        compiler_params=pltpu.CompilerParams(collective_id=0),
    )(x, rT, W1, W2)
